# baseline (speedup 1.0000x reference)
import functools

import jax
import jax.numpy as jnp
from jax import lax
from jax.experimental import pallas as pl
from jax.experimental.pallas import tpu as pltpu
from jax.experimental.pallas import tpu_sc as plsc

_B, _C, _D = 8, 1024, 4096
_STRIDE = 8
_K = _C // _STRIDE
_NC, _NS = 2, 16
_NW = _NC * _NS
_ROWS = (_B * _K) // _NW
_CH = 8
_NCH = _ROWS // _CH
_NSLOT = 3


def _copy_body(x_hbm, out_hbm, buf, sem_in, sem_out):
    wid = lax.axis_index("s") * _NC + lax.axis_index("c")
    b = wid // 4
    c_base = (wid % 4) * _ROWS

    def start_in(j):
        return pltpu.async_copy(
            x_hbm.at[b, pl.ds(c_base + j * _CH, _CH), 0, :],
            buf.at[j % _NSLOT], sem_in)

    def start_out(j):
        return pltpu.async_copy(
            buf.at[j % _NSLOT],
            out_hbm.at[b, pl.ds(c_base + j * _CH, _CH), :], sem_out)

    cin = [None] * _NCH
    cout = [None] * _NCH
    for j in range(min(_NSLOT - 1, _NCH)):
        cin[j] = start_in(j)
    for j in range(_NCH):
        nxt = j + _NSLOT - 1
        if nxt < _NCH:
            if j >= 1:
                cout[j - 1].wait()
            cin[nxt] = start_in(nxt)
        cin[j].wait()
        cout[j] = start_out(j)
    for j in range(max(0, _NCH - _NSLOT), _NCH):
        cout[j].wait()


@jax.jit
def _channel_select(x):
    xv = x.reshape(_B, _K, _STRIDE, _D)
    mesh = plsc.VectorSubcoreMesh(core_axis_name="c", subcore_axis_name="s")
    run = functools.partial(
        pl.kernel,
        mesh=mesh,
        out_type=jax.ShapeDtypeStruct((_B, _K, _D), jnp.float32),
        scratch_types=[
            pltpu.VMEM((_NSLOT, _CH, _D), jnp.float32),
            pltpu.SemaphoreType.DMA,
            pltpu.SemaphoreType.DMA,
        ],
    )(_copy_body)
    return run(xv)


def kernel(x):
    return _channel_select(x)


# speedup vs baseline: 2.4505x; 2.4505x over previous
import functools

import jax
import jax.numpy as jnp
from jax import lax
from jax.experimental import pallas as pl
from jax.experimental.pallas import tpu as pltpu
from jax.experimental.pallas import tpu_sc as plsc

_B, _C, _D = 8, 1024, 4096
_STRIDE = 8
_K = _C // _STRIDE
_NC, _NS = 2, 16
_NW = _NC * _NS
_ROWS = (_B * _K) // _NW
_CH = 4
_NCH = _ROWS // _CH
_NSLOT = 7


def _copy_body(x_hbm, out_hbm, buf, sem_in, sem_out):
    wid = lax.axis_index("s") * _NC + lax.axis_index("c")
    b = wid // 4
    c_base = (wid % 4) * _ROWS

    def start_in(j):
        return pltpu.async_copy(
            x_hbm.at[b, pl.ds(c_base + j * _CH, _CH), 0, :],
            buf.at[j % _NSLOT], sem_in)

    def start_out(j):
        return pltpu.async_copy(
            buf.at[j % _NSLOT],
            out_hbm.at[b, pl.ds(c_base + j * _CH, _CH), :], sem_out)

    cin = [None] * _NCH
    cout = [None] * _NCH
    for j in range(min(_NSLOT - 1, _NCH)):
        cin[j] = start_in(j)
    for j in range(_NCH):
        nxt = j + _NSLOT - 1
        if nxt < _NCH:
            if j >= 1:
                cout[j - 1].wait()
            cin[nxt] = start_in(nxt)
        cin[j].wait()
        cout[j] = start_out(j)
    for j in range(max(0, _NCH - _NSLOT), _NCH):
        cout[j].wait()


@jax.jit
def _channel_select(x):
    xv = x.reshape(_B, _K, _STRIDE, _D)
    mesh = plsc.VectorSubcoreMesh(core_axis_name="c", subcore_axis_name="s")
    run = functools.partial(
        pl.kernel,
        mesh=mesh,
        out_type=jax.ShapeDtypeStruct((_B, _K, _D), jnp.float32),
        scratch_types=[
            pltpu.VMEM((_NSLOT, _CH, _D), jnp.float32),
            pltpu.SemaphoreType.DMA,
            pltpu.SemaphoreType.DMA,
        ],
    )(_copy_body)
    return run(xv)


_TC_SLOT = 4


def _tc_copy_body(x_hbm, o_hbm, buf, sem_in, sem_out):
    def start_in(j):
        return pltpu.make_async_copy(
            x_hbm.at[j, :, 0, :], buf.at[j % _TC_SLOT], sem_in)

    def start_out(j):
        return pltpu.make_async_copy(
            buf.at[j % _TC_SLOT], o_hbm.at[j], sem_out)

    cin = [None] * _B
    cout = [None] * _B
    for j in range(min(_TC_SLOT - 1, _B)):
        cin[j] = start_in(j)
        cin[j].start()
    for j in range(_B):
        nxt = j + _TC_SLOT - 1
        if nxt < _B:
            if j >= 1:
                cout[j - 1].wait()
            cin[nxt] = start_in(nxt)
            cin[nxt].start()
        cin[j].wait()
        cout[j] = start_out(j)
        cout[j].start()
    for j in range(max(0, _B - _TC_SLOT), _B):
        cout[j].wait()


@jax.jit
def _channel_select_tc(x):
    xv = x.reshape(_B, _K, _STRIDE, _D)
    return pl.pallas_call(
        _tc_copy_body,
        in_specs=[pl.BlockSpec(memory_space=pl.ANY)],
        out_specs=pl.BlockSpec(memory_space=pl.ANY),
        out_shape=jax.ShapeDtypeStruct((_B, _K, _D), jnp.float32),
        scratch_shapes=[
            pltpu.VMEM((_TC_SLOT, _K, _D), jnp.float32),
            pltpu.SemaphoreType.DMA,
            pltpu.SemaphoreType.DMA,
        ],
    )(xv)


def kernel(x):
    return _channel_select_tc(x)
